# baseline (speedup 1.0000x reference)
import functools

import jax
import jax.numpy as jnp
from jax import lax
from jax.experimental import pallas as pl
from jax.experimental.pallas import tpu as pltpu
from jax.experimental.pallas import tpu_sc as plsc

NUM_CODES = 1024
LATENT_DIM = 256
TOK_BLOCK = 1024
NTOK = 8192

_SC_INFO = plsc.get_sparse_core_info()
_NW = _SC_INFO.num_cores * _SC_INFO.num_subcores
_ROWS_PER_W = NTOK // _NW


def _vq_body(zb_ref, wt_ref, sw_ref, idx_ref):
    zb = zb_ref[...]
    wt = wt_ref[...]
    sz = jnp.sum(zb * zb, axis=1, keepdims=True)
    mm = jnp.dot(zb, wt, preferred_element_type=jnp.float32)
    d = (sz + sw_ref[...]) - 2.0 * mm
    minv = jnp.min(d, axis=1, keepdims=True)
    iota = jax.lax.broadcasted_iota(jnp.int32, (TOK_BLOCK, NUM_CODES), 1)
    idx = jnp.min(jnp.where(d == minv, iota, NUM_CODES), axis=1)
    idx_ref[0] = idx[None, :]


def _sc_gather(table_hbm, idx_hbm, out_hbm, idx_v, rows_v, sem):
    wid = lax.axis_index("s") * _SC_INFO.num_cores + lax.axis_index("c")
    base = wid * _ROWS_PER_W
    pltpu.sync_copy(idx_hbm.at[pl.ds(base, _ROWS_PER_W)], idx_v)
    pltpu.async_copy(table_hbm.at[idx_v], rows_v, sem).wait()
    pltpu.sync_copy(rows_v, out_hbm.at[pl.ds(base, _ROWS_PER_W)])


def kernel(z, W):
    B, C, H, Wd = z.shape
    S = H * Wd
    ntok = B * S
    nblk = ntok // TOK_BLOCK
    zp = jnp.transpose(z, (0, 2, 3, 1)).reshape(ntok, LATENT_DIM)
    sw = jnp.sum(W ** 2, axis=1).reshape(1, NUM_CODES)
    wt = W.T

    idx3d = pl.pallas_call(
        _vq_body,
        grid=(nblk,),
        in_specs=[
            pl.BlockSpec((TOK_BLOCK, LATENT_DIM), lambda i: (i, 0)),
            pl.BlockSpec((LATENT_DIM, NUM_CODES), lambda i: (0, 0)),
            pl.BlockSpec((1, NUM_CODES), lambda i: (0, 0)),
        ],
        out_specs=pl.BlockSpec((1, 1, TOK_BLOCK), lambda i: (i, 0, 0)),
        out_shape=jax.ShapeDtypeStruct((nblk, 1, TOK_BLOCK), jnp.int32),
    )(zp, wt, sw)
    indices = idx3d.reshape(ntok)

    gather = functools.partial(
        pl.kernel,
        mesh=plsc.VectorSubcoreMesh(core_axis_name="c", subcore_axis_name="s"),
        out_type=jax.ShapeDtypeStruct((ntok, LATENT_DIM), jnp.float32),
        scratch_types=[
            pltpu.VMEM((_ROWS_PER_W,), jnp.int32),
            pltpu.VMEM((_ROWS_PER_W, LATENT_DIM), jnp.float32),
            pltpu.SemaphoreType.DMA,
        ],
    )(_sc_gather)
    zq_flat = gather(W, indices)

    z_q = jnp.transpose(zq_flat.reshape(B, H, Wd, LATENT_DIM), (0, 3, 1, 2))
    return (indices, z_q)


# speedup vs baseline: 1.1084x; 1.1084x over previous
import jax
import jax.numpy as jnp
from jax.experimental import pallas as pl

NUM_CODES = 1024
LATENT_DIM = 256
TOK_BLOCK = 1024


def _vq_body(zbt_ref, wt_ref, sw_ref, idx_ref, zq_ref):
    zb = jnp.transpose(zbt_ref[0], (1, 0))
    wt = wt_ref[...]
    sz = jnp.sum(zb * zb, axis=1, keepdims=True)
    mm = jnp.dot(zb, wt, preferred_element_type=jnp.float32)
    d = (sz + sw_ref[...]) - 2.0 * mm
    minv = jnp.min(d, axis=1, keepdims=True)
    iota = jax.lax.broadcasted_iota(jnp.int32, (TOK_BLOCK, NUM_CODES), 1)
    idx = jnp.min(jnp.where(d == minv, iota, NUM_CODES), axis=1)
    idx_ref[0] = idx[None, :]
    iota_t = jax.lax.broadcasted_iota(jnp.int32, (NUM_CODES, TOK_BLOCK), 0)
    onehot = (iota_t == idx[None, :]).astype(jnp.float32)
    zq_ref[0] = jax.lax.dot_general(
        wt, onehot, (((1,), (0,)), ((), ())),
        preferred_element_type=jnp.float32)


def kernel(z, W):
    B, C, H, Wd = z.shape
    S = H * Wd
    ntok = B * S
    nblk = ntok // TOK_BLOCK
    z3 = z.reshape(B, C, S)
    sw = jnp.sum(W ** 2, axis=1).reshape(1, NUM_CODES)
    wt = W.T

    idx3d, zqt = pl.pallas_call(
        _vq_body,
        grid=(nblk,),
        in_specs=[
            pl.BlockSpec((1, LATENT_DIM, TOK_BLOCK), lambda i: (i, 0, 0)),
            pl.BlockSpec((LATENT_DIM, NUM_CODES), lambda i: (0, 0)),
            pl.BlockSpec((1, NUM_CODES), lambda i: (0, 0)),
        ],
        out_specs=[
            pl.BlockSpec((1, 1, TOK_BLOCK), lambda i: (i, 0, 0)),
            pl.BlockSpec((1, LATENT_DIM, TOK_BLOCK), lambda i: (i, 0, 0)),
        ],
        out_shape=[
            jax.ShapeDtypeStruct((nblk, 1, TOK_BLOCK), jnp.int32),
            jax.ShapeDtypeStruct((nblk, LATENT_DIM, TOK_BLOCK), jnp.float32),
        ],
    )(z3, wt, sw)

    indices = idx3d.reshape(ntok)
    z_q = zqt.reshape(B, LATENT_DIM, H, Wd)
    return (indices, z_q)


# speedup vs baseline: 1.1962x; 1.0792x over previous
import jax
import jax.numpy as jnp
from jax.experimental import pallas as pl

NUM_CODES = 1024
LATENT_DIM = 256
TOK_BLOCK = 1024


def _vq_body(zbt_ref, w_ref, wt_ref, swc_ref, idx_ref, zq_ref):
    zbt = zbt_ref[0]
    w = w_ref[...]
    wt = wt_ref[...]
    szt = jnp.sum(zbt * zbt, axis=0, keepdims=True)
    mmt = jax.lax.dot_general(
        w, zbt, (((1,), (0,)), ((), ())),
        preferred_element_type=jnp.float32)
    d = (szt + swc_ref[...]) - 2.0 * mmt
    minv = jnp.min(d, axis=0, keepdims=True)
    iota_t = jax.lax.broadcasted_iota(jnp.int32, (NUM_CODES, TOK_BLOCK), 0)
    idx = jnp.min(jnp.where(d == minv, iota_t, NUM_CODES), axis=0)
    idx_ref[0] = idx[None, :]
    onehot = (iota_t == idx[None, :]).astype(jnp.float32)
    zq_ref[0] = jax.lax.dot_general(
        wt, onehot, (((1,), (0,)), ((), ())),
        preferred_element_type=jnp.float32)


def kernel(z, W):
    B, C, H, Wd = z.shape
    S = H * Wd
    ntok = B * S
    nblk = ntok // TOK_BLOCK
    z3 = z.reshape(B, C, S)
    swc = jnp.sum(W ** 2, axis=1).reshape(NUM_CODES, 1)
    wt = W.T

    idx3d, zqt = pl.pallas_call(
        _vq_body,
        grid=(nblk,),
        in_specs=[
            pl.BlockSpec((1, LATENT_DIM, TOK_BLOCK), lambda i: (i, 0, 0)),
            pl.BlockSpec((NUM_CODES, LATENT_DIM), lambda i: (0, 0)),
            pl.BlockSpec((LATENT_DIM, NUM_CODES), lambda i: (0, 0)),
            pl.BlockSpec((NUM_CODES, 1), lambda i: (0, 0)),
        ],
        out_specs=[
            pl.BlockSpec((1, 1, TOK_BLOCK), lambda i: (i, 0, 0)),
            pl.BlockSpec((1, LATENT_DIM, TOK_BLOCK), lambda i: (i, 0, 0)),
        ],
        out_shape=[
            jax.ShapeDtypeStruct((nblk, 1, TOK_BLOCK), jnp.int32),
            jax.ShapeDtypeStruct((nblk, LATENT_DIM, TOK_BLOCK), jnp.float32),
        ],
    )(z3, W, wt, swc)

    indices = idx3d.reshape(ntok)
    z_q = zqt.reshape(B, LATENT_DIM, H, Wd)
    return (indices, z_q)


# speedup vs baseline: 1.2532x; 1.0476x over previous
import jax
import jax.numpy as jnp
from jax.experimental import pallas as pl

NUM_CODES = 1024
LATENT_DIM = 256
TOK_BLOCK = 512


def _vq_body(zb_ref, wt_ref, sw_ref, idx_ref, zq_ref):
    zb = zb_ref[...]
    wt = wt_ref[...]
    sz = jnp.sum(zb * zb, axis=1, keepdims=True)
    mm = jnp.dot(zb, wt, preferred_element_type=jnp.float32)
    d = (sz + sw_ref[...]) - 2.0 * mm
    minv = jnp.min(d, axis=1, keepdims=True)
    iota = jax.lax.broadcasted_iota(jnp.int32, (TOK_BLOCK, NUM_CODES), 1)
    idx = jnp.min(jnp.where(d == minv, iota, NUM_CODES), axis=1)
    idx_ref[0] = idx[None, :]
    iota_t = jax.lax.broadcasted_iota(jnp.int32, (NUM_CODES, TOK_BLOCK), 0)
    onehot = (iota_t == idx[None, :]).astype(jnp.float32)
    zq_ref[0] = jax.lax.dot_general(
        wt, onehot, (((1,), (0,)), ((), ())),
        preferred_element_type=jnp.float32)


def kernel(z, W):
    B, C, H, Wd = z.shape
    S = H * Wd
    ntok = B * S
    nblk = ntok // TOK_BLOCK
    zp = jnp.transpose(z, (0, 2, 3, 1)).reshape(ntok, LATENT_DIM)
    sw = jnp.sum(W ** 2, axis=1).reshape(1, NUM_CODES)
    wt = W.T

    per_b = S // TOK_BLOCK
    idx3d, zqt = pl.pallas_call(
        _vq_body,
        grid=(nblk,),
        in_specs=[
            pl.BlockSpec((TOK_BLOCK, LATENT_DIM), lambda i: (i, 0)),
            pl.BlockSpec((LATENT_DIM, NUM_CODES), lambda i: (0, 0)),
            pl.BlockSpec((1, NUM_CODES), lambda i: (0, 0)),
        ],
        out_specs=[
            pl.BlockSpec((1, 1, TOK_BLOCK), lambda i: (i, 0, 0)),
            pl.BlockSpec((1, LATENT_DIM, TOK_BLOCK),
                         lambda i: (i // per_b, 0, i % per_b)),
        ],
        out_shape=[
            jax.ShapeDtypeStruct((nblk, 1, TOK_BLOCK), jnp.int32),
            jax.ShapeDtypeStruct((B, LATENT_DIM, S), jnp.float32),
        ],
    )(zp, wt, sw)

    indices = idx3d.reshape(ntok)
    z_q = zqt.reshape(B, LATENT_DIM, H, Wd)
    return (indices, z_q)


# speedup vs baseline: 1.4940x; 1.1921x over previous
import jax
import jax.numpy as jnp
from jax.experimental import pallas as pl

NUM_CODES = 1024
LATENT_DIM = 256
TOK_BLOCK = 2048
BATCHES_PER_BLOCK = 2


def _vq_body(zb_ref, wt_ref, sw_ref, idx_ref, zq_ref):
    zb = zb_ref[...]
    wt = wt_ref[...]
    sz = jnp.sum(zb * zb, axis=1, keepdims=True)
    mm = jnp.dot(zb, wt, preferred_element_type=jnp.float32)
    d = (sz + sw_ref[...]) - 2.0 * mm
    minv = jnp.min(d, axis=1, keepdims=True)
    iota = jax.lax.broadcasted_iota(jnp.int32, (TOK_BLOCK, NUM_CODES), 1)
    idx = jnp.min(jnp.where(d == minv, iota, NUM_CODES), axis=1)
    idx_ref[0] = idx[None, :]
    iota_t = jax.lax.broadcasted_iota(jnp.int32, (NUM_CODES, TOK_BLOCK), 0)
    onehot = (iota_t == idx[None, :]).astype(jnp.float32)
    zq = jax.lax.dot_general(
        wt, onehot, (((1,), (0,)), ((), ())),
        preferred_element_type=jnp.float32)
    sp = TOK_BLOCK // BATCHES_PER_BLOCK
    for j in range(BATCHES_PER_BLOCK):
        zq_ref[j] = zq[:, j * sp:(j + 1) * sp]


def kernel(z, W):
    B, C, H, Wd = z.shape
    S = H * Wd
    ntok = B * S
    nblk = ntok // TOK_BLOCK
    zp = jnp.transpose(z, (0, 2, 3, 1)).reshape(ntok, LATENT_DIM)
    sw = jnp.sum(W ** 2, axis=1).reshape(1, NUM_CODES)
    wt = W.T

    idx2d, zqt = pl.pallas_call(
        _vq_body,
        grid=(nblk,),
        in_specs=[
            pl.BlockSpec((TOK_BLOCK, LATENT_DIM), lambda i: (i, 0)),
            pl.BlockSpec((LATENT_DIM, NUM_CODES), lambda i: (0, 0)),
            pl.BlockSpec((1, NUM_CODES), lambda i: (0, 0)),
        ],
        out_specs=[
            pl.BlockSpec((1, 1, TOK_BLOCK), lambda i: (i, 0, 0)),
            pl.BlockSpec((BATCHES_PER_BLOCK, LATENT_DIM, S),
                         lambda i: (i, 0, 0)),
        ],
        out_shape=[
            jax.ShapeDtypeStruct((nblk, 1, TOK_BLOCK), jnp.int32),
            jax.ShapeDtypeStruct((B, LATENT_DIM, S), jnp.float32),
        ],
    )(zp, wt, sw)

    indices = idx2d.reshape(ntok)
    z_q = zqt.reshape(B, LATENT_DIM, H, Wd)
    return (indices, z_q)
